# baseline (speedup 1.0000x reference)
import jax
import jax.numpy as jnp
from jax.experimental import pallas as pl

NUM_USERS = 30000
N_NODES = 50000
NUM_LAYERS = 3


def _scale_body(x_ref, o_ref):
    o_ref[...] = x_ref[...] * 0.25


def kernel(user_emb, artist_emb, album_emb, item_audio_emb, audio_proj_W,
           mlp_W1, mlp_b1, mlp_W2, mlp_b2, edge_features,
           artist_ids, album_ids, adjusted_edge_index):
    user_h = user_emb
    item_h = item_audio_emb + artist_emb[artist_ids] + album_emb[album_ids]
    projected_audio = item_audio_emb @ audio_proj_W.T
    h = jax.nn.relu(edge_features @ mlp_W1.T + mlp_b1)
    edge_weight = jax.nn.sigmoid(h @ mlp_W2.T + mlp_b2).squeeze(-1)
    row = adjusted_edge_index[0]
    col = adjusted_edge_index[1]
    x = jnp.concatenate([user_h, item_h], axis=0)
    deg = jnp.zeros((N_NODES,), x.dtype).at[col].add(edge_weight)
    safe = jnp.where(deg > 0, deg, 1.0)
    dis = jnp.where(deg > 0, safe ** -0.5, 0.0)
    norm = dis[row] * edge_weight * dis[col]
    acc = x
    for _ in range(NUM_LAYERS):
        msg = x[row] * norm[:, None]
        x = jnp.zeros_like(x).at[col].add(msg)
        acc = acc + x
    x_final = pl.pallas_call(
        _scale_body,
        out_shape=jax.ShapeDtypeStruct(acc.shape, acc.dtype),
    )(acc)
    final_user_h = x_final[:NUM_USERS]
    final_item_h = x_final[NUM_USERS:]
    align_loss = jnp.mean((final_item_h - projected_audio) ** 2)
    return (final_user_h, final_item_h, align_loss)


# speedup vs baseline: 7.9148x; 7.9148x over previous
import functools

import jax
import jax.numpy as jnp
from jax import lax
from jax.experimental import pallas as pl
from jax.experimental.pallas import tpu as pltpu
from jax.experimental.pallas import tpu_sc as plsc

NUM_USERS = 30000
NUM_ITEMS = 20000
N_NODES = 50000
E = 800000
D = 64
HD = 32
NUM_LAYERS = 3

ER = 6656
EP = ER * 128
ROWS_PER_TILE = ER // 16
SUPERS = ROWS_PER_TILE // 32

DEG_ROWS = 512
DEG_TROWS = DEG_ROWS // 16

ITEM_CHUNK = 80
N_ITEM_CHUNKS = NUM_ITEMS // ITEM_CHUNK

_mesh = plsc.VectorSubcoreMesh(core_axis_name="c", subcore_axis_name="s")


def _rsqrt16(x):
    i = plsc.bitcast(x, jnp.int32)
    i = jnp.int32(0x5F3759DF) - (i >> 1)
    y = plsc.bitcast(i, jnp.float32)
    for _ in range(3):
        y = y * (1.5 - 0.5 * x * y * y)
    return jnp.where(x > 0.0, y, 0.0)


def _mlp_body(ef_ref, w1_ref, b1_ref, w2_ref, b2_ref, out_ref):
    ef = ef_ref[...]
    w1 = w1_ref[...]
    h = lax.dot_general(w1, ef, (((1,), (0,)), ((), ())),
                        preferred_element_type=jnp.float32)
    h = jnp.maximum(h + b1_ref[...], 0.0)
    z = lax.dot_general(w2_ref[...], h, (((1,), (0,)), ((), ())),
                        preferred_element_type=jnp.float32)
    out_ref[...] = jax.nn.sigmoid(z + b2_ref[...])


def _edge_mlp(ef_t, w1, b1, w2, b2):
    B = 80000
    grid = E // B
    return pl.pallas_call(
        _mlp_body,
        grid=(grid,),
        in_specs=[
            pl.BlockSpec((5, B), lambda i: (0, i)),
            pl.BlockSpec((32, 5), lambda i: (0, 0)),
            pl.BlockSpec((32, 1), lambda i: (0, 0)),
            pl.BlockSpec((1, 32), lambda i: (0, 0)),
            pl.BlockSpec((1, 1), lambda i: (0, 0)),
        ],
        out_specs=pl.BlockSpec((1, B), lambda i: (0, i)),
        out_shape=jax.ShapeDtypeStruct((1, E), jnp.float32),
    )(ef_t, w1, b1.reshape(32, 1), w2, b2.reshape(1, 1))


def _prep_body(artist_hbm, album_hbm, audio_hbm, aid_hbm, bid_hbm,
               row_hbm, col_hbm, ew_hbm,
               item_hbm, norm_hbm,
               aid_v, bid_v, abuf, bbuf, cbuf,
               deg_tile, idxv, ewv, rowv, nbuf, dbuf, didx,
               deg_sh, dis_sh,
               sem_a, sem_b, sem_c):
    cid = lax.axis_index("c")
    sid = lax.axis_index("s")
    wid = sid * 2 + cid

    def item_chunk(k, _):
        c = wid + 32 * k

        @pl.when(c < N_ITEM_CHUNKS)
        def _():
            base = c * ITEM_CHUNK
            pltpu.sync_copy(aid_hbm.at[pl.ds(base, ITEM_CHUNK)], aid_v)
            pltpu.sync_copy(bid_hbm.at[pl.ds(base, ITEM_CHUNK)], bid_v)
            da = pltpu.async_copy(artist_hbm.at[aid_v], abuf, sem_a)
            db = pltpu.async_copy(album_hbm.at[bid_v], bbuf, sem_b)
            dc = pltpu.async_copy(audio_hbm.at[pl.ds(base, ITEM_CHUNK)], cbuf,
                                  sem_c)
            da.wait()
            db.wait()
            dc.wait()

            def add_row(r, _):
                for kk in range(4):
                    sl = pl.ds(kk * 16, 16)
                    abuf[r, sl] = abuf[r, sl] + bbuf[r, sl] + cbuf[r, sl]
                return ()

            lax.fori_loop(0, ITEM_CHUNK, add_row, (), unroll=4)
            pltpu.sync_copy(abuf, item_hbm.at[pl.ds(base, ITEM_CHUNK)])

        return ()

    lax.fori_loop(0, 8, item_chunk, ())

    def zero_deg(r, _):
        for kk in range(8):
            deg_tile[r, pl.ds(kk * 16, 16)] = jnp.zeros((16,), jnp.float32)
        return ()

    lax.fori_loop(0, DEG_ROWS, zero_deg, (), unroll=2)
    pltpu.sync_copy(deg_tile.at[pl.ds(sid * DEG_TROWS, DEG_TROWS)],
                    deg_sh.at[pl.ds(sid * DEG_TROWS, DEG_TROWS)])

    for j in range(4):
        for kk in range(8):
            didx[j, pl.ds(kk * 16, 16)] = (
                jnp.arange(16, dtype=jnp.int32) + (j * 128 + kk * 16))

    r0_tile = sid * ROWS_PER_TILE

    def deg_super(sb, _):
        r0 = r0_tile + sb * 32
        pltpu.sync_copy(col_hbm.at[pl.ds(r0, 32)], idxv)
        pltpu.sync_copy(ew_hbm.at[pl.ds(r0, 32)], ewv)

        def deg_row(j, _):
            for kk in range(8):
                sl = pl.ds(kk * 16, 16)
                c16 = idxv[j, sl]
                plsc.addupdate_scatter(
                    deg_tile, [c16 >> 7, c16 & 127], ewv[j, sl])
            return ()

        lax.fori_loop(0, 32, deg_row, ())
        return ()

    lax.fori_loop(0, SUPERS, deg_super, ())
    plsc.subcore_barrier()
    for j in range(4):
        pltpu.sync_copy(deg_tile.at[pl.ds(j * 128, 128)],
                        deg_sh.at[didx.at[j]], add=True)
    plsc.subcore_barrier()

    pltpu.sync_copy(deg_sh.at[pl.ds(sid * DEG_TROWS, DEG_TROWS)], dbuf)

    def dis_group(r, _):
        for kk in range(8):
            sl = pl.ds(kk * 16, 16)
            dbuf[r, sl] = _rsqrt16(dbuf[r, sl])
        return ()

    lax.fori_loop(0, DEG_TROWS, dis_group, ())
    pltpu.sync_copy(dbuf, dis_sh.at[pl.ds(sid * DEG_TROWS, DEG_TROWS)])
    plsc.subcore_barrier()

    pltpu.sync_copy(dis_sh, deg_tile)

    def norm_super(sb, _):
        r0 = r0_tile + sb * 32
        pltpu.sync_copy(row_hbm.at[pl.ds(r0, 32)], rowv)
        pltpu.sync_copy(col_hbm.at[pl.ds(r0, 32)], idxv)
        pltpu.sync_copy(ew_hbm.at[pl.ds(r0, 32)], ewv)

        def norm_row(j, _):
            for kk in range(8):
                sl = pl.ds(kk * 16, 16)
                r16 = rowv[j, sl]
                c16 = idxv[j, sl]
                dr = plsc.load_gather(deg_tile, [r16 >> 7, r16 & 127])
                dc = plsc.load_gather(deg_tile, [c16 >> 7, c16 & 127])
                nbuf[j, sl] = dr * ewv[j, sl] * dc
            return ()

        lax.fori_loop(0, 32, norm_row, ())
        pltpu.sync_copy(nbuf, norm_hbm.at[pl.ds(r0, 32)])
        return ()

    lax.fori_loop(0, SUPERS, norm_super, ())


def _sc_prep(artist_emb, album_emb, item_audio_emb, artist_ids, album_ids,
             row2d, col2d, ew2d):
    f32 = jnp.float32
    i32 = jnp.int32
    kern = pl.kernel(
        _prep_body,
        out_type=(
            jax.ShapeDtypeStruct((NUM_ITEMS, D), f32),
            jax.ShapeDtypeStruct((ER, 128), f32),
        ),
        mesh=_mesh,
        compiler_params=pltpu.CompilerParams(needs_layout_passes=False,
                                             use_tc_tiling_on_sc=False),
        scratch_types=[
            pltpu.VMEM((ITEM_CHUNK,), i32),
            pltpu.VMEM((ITEM_CHUNK,), i32),
            pltpu.VMEM((ITEM_CHUNK, D), f32),
            pltpu.VMEM((ITEM_CHUNK, D), f32),
            pltpu.VMEM((ITEM_CHUNK, D), f32),
            pltpu.VMEM((DEG_ROWS, 128), f32),
            pltpu.VMEM((32, 128), i32),
            pltpu.VMEM((32, 128), f32),
            pltpu.VMEM((32, 128), i32),
            pltpu.VMEM((32, 128), f32),
            pltpu.VMEM((DEG_TROWS, 128), f32),
            pltpu.VMEM((4, 128), i32),
            pltpu.VMEM_SHARED((DEG_ROWS, 128), f32),
            pltpu.VMEM_SHARED((DEG_ROWS, 128), f32),
            pltpu.SemaphoreType.DMA,
            pltpu.SemaphoreType.DMA,
            pltpu.SemaphoreType.DMA,
        ],
    )
    return kern(artist_emb, album_emb, item_audio_emb, artist_ids, album_ids,
                row2d, col2d, ew2d)


def _layer_body(xs_hbm, rowf_hbm, col_hbm, normf_hbm, ys_hbm,
                rows, rowi, coli, normv,
                accum, gsem0, gsem1, zsem):
    cid = lax.axis_index("c")
    sid = lax.axis_index("s")
    gsems = (gsem0, gsem1)
    xoff = cid * N_NODES

    def zrow(r, _):
        for kk in range(2):
            rows[0, r, pl.ds(kk * 16, 16)] = jnp.zeros((16,), jnp.float32)
        return ()
    lax.fori_loop(0, 256, zrow, (), unroll=4)

    zbase = sid * (N_NODES // 16)

    zdescs = []
    for t in range(12):
        zdescs.append(pltpu.async_copy(
            rows.at[0], accum.at[pl.ds(zbase + t * 256, 256)], zsem))
    zdescs.append(pltpu.async_copy(
        rows.at[0, pl.ds(0, 53)], accum.at[pl.ds(zbase + 3072, 53)], zsem))
    for d in zdescs:
        d.wait()
    plsc.subcore_barrier()

    e0_tile = sid * ROWS_PER_TILE * 128

    def _gather(g, b):
        return pltpu.async_copy(
            xs_hbm.at[rowi.at[pl.ds(g * 256, 256)]], rows.at[b], gsems[b])

    def super_chunk(sb, _):
        e0 = e0_tile + sb * 4096
        r0 = (e0 >> 7)
        pltpu.sync_copy(rowf_hbm.at[pl.ds(e0, 4096)], rowi)
        pltpu.sync_copy(col_hbm.at[pl.ds(r0, 32)], coli)
        pltpu.sync_copy(normf_hbm.at[pl.ds(e0, 4096)], normv)

        def adj_row(j, _):
            rowi[pl.ds(j * 16, 16)] = rowi[pl.ds(j * 16, 16)] + xoff
            return ()

        lax.fori_loop(0, 256, adj_row, (), unroll=8)

        _gather(0, 0)
        _gather(1, 1)
        for g in range(16):
            b = g & 1
            pltpu.make_async_copy(
                xs_hbm.at[rowi.at[pl.ds(g * 256, 256)]], rows.at[b],
                gsems[b]).wait()

            def scale(q, _):
                nv = normv[pl.ds(g * 256 + q * 16, 16)]
                for i in range(16):
                    e = q * 16 + i
                    s = nv[i]
                    for kk in range(2):
                        sl = pl.ds(kk * 16, 16)
                        rows[b, e, sl] = rows[b, e, sl] * s
                return ()

            lax.fori_loop(0, 16, scale, ())
            for j in range(2):
                pltpu.sync_copy(rows.at[b, pl.ds(j * 128, 128)],
                                accum.at[coli.at[g * 2 + j]], add=True)
            if g + 2 < 16:
                _gather(g + 2, b)
        return ()

    lax.fori_loop(0, SUPERS, super_chunk, ())
    plsc.subcore_barrier()

    obase = sid * (N_NODES // 16)
    pltpu.sync_copy(accum.at[pl.ds(obase, N_NODES // 16)],
                    ys_hbm.at[pl.ds(xoff + obase, N_NODES // 16)])


def _sc_layer(xs, rowf, col2d, normf):
    f32 = jnp.float32
    i32 = jnp.int32
    kern = pl.kernel(
        _layer_body,
        out_type=jax.ShapeDtypeStruct((2 * N_NODES, HD), f32),
        mesh=_mesh,
        compiler_params=pltpu.CompilerParams(needs_layout_passes=False,
                                             use_tc_tiling_on_sc=False),
        scratch_types=[
            pltpu.VMEM((2, 256, HD), f32),
            pltpu.VMEM((4096,), i32),
            pltpu.VMEM((32, 128), i32),
            pltpu.VMEM((4096,), f32),
            pltpu.VMEM_SHARED((N_NODES, HD), f32),
            pltpu.SemaphoreType.DMA,
            pltpu.SemaphoreType.DMA,
            pltpu.SemaphoreType.DMA,
        ],
    )
    return kern(xs, rowf, col2d, normf)


def _final_body(x0_ref, aL_ref, aR_ref, bL_ref, bR_ref, cL_ref, cR_ref,
                ia_ref, w_ref, out_ref, loss_ref, acc_ref):
    i = pl.program_id(0)
    x0 = x0_ref[...]
    fL = (x0[:, :HD] + aL_ref[...] + bL_ref[...] + cL_ref[...]) * 0.25
    fR = (x0[:, HD:] + aR_ref[...] + bR_ref[...] + cR_ref[...]) * 0.25
    f = jnp.concatenate([fL, fR], axis=1)
    out_ref[...] = f

    @pl.when(i == 0)
    def _():
        acc_ref[0] = 0.0

    @pl.when(i >= 15)
    def _():
        proj = lax.dot_general(ia_ref[...], w_ref[...], (((1,), (1,)), ((), ())),
                               preferred_element_type=jnp.float32)
        d = f - proj
        acc_ref[0] += jnp.sum(d * d)

    @pl.when(i == 24)
    def _():
        loss_ref[...] = jnp.full((1, 1), acc_ref[0] / float(NUM_ITEMS * D),
                                 jnp.float32)


def _tc_final(x0, x1s, x2s, x3s, item_audio_emb, audio_proj_W):
    B = 2000
    nU = NUM_USERS // B
    half = pl.BlockSpec((B, HD), lambda i: (i, 0))

    def rhalf(i):
        return (i + N_NODES // B, 0)

    out, loss = pl.pallas_call(
        _final_body,
        grid=(N_NODES // B,),
        in_specs=[
            pl.BlockSpec((B, D), lambda i: (i, 0)),
            half, pl.BlockSpec((B, HD), rhalf),
            half, pl.BlockSpec((B, HD), rhalf),
            half, pl.BlockSpec((B, HD), rhalf),
            pl.BlockSpec((B, D), lambda i: (jnp.maximum(i - nU, 0), 0)),
            pl.BlockSpec((D, D), lambda i: (0, 0)),
        ],
        out_specs=[
            pl.BlockSpec((B, D), lambda i: (i, 0)),
            pl.BlockSpec((1, 1), lambda i: (0, 0)),
        ],
        out_shape=[
            jax.ShapeDtypeStruct((N_NODES, D), jnp.float32),
            jax.ShapeDtypeStruct((1, 1), jnp.float32),
        ],
        scratch_shapes=[pltpu.SMEM((1,), jnp.float32)],
    )(x0, x1s, x1s, x2s, x2s, x3s, x3s, item_audio_emb, audio_proj_W)
    return out, loss


@jax.jit
def kernel(user_emb, artist_emb, album_emb, item_audio_emb, audio_proj_W,
           mlp_W1, mlp_b1, mlp_W2, mlp_b2, edge_features,
           artist_ids, album_ids, adjusted_edge_index):
    i32 = jnp.int32
    row = adjusted_edge_index[0].astype(i32)
    col = adjusted_edge_index[1].astype(i32)
    aid = artist_ids.astype(i32)
    bid = album_ids.astype(i32)

    ef_t = edge_features.T
    ew = _edge_mlp(ef_t, mlp_W1, mlp_b1, mlp_W2, mlp_b2)

    pad = EP - E
    row2d = jnp.pad(row, (0, pad)).reshape(ER, 128)
    col2d = jnp.pad(col, (0, pad)).reshape(ER, 128)
    ew2d = jnp.pad(ew[0], (0, pad)).reshape(ER, 128)

    item_h, norm2d = _sc_prep(artist_emb, album_emb, item_audio_emb,
                              aid, bid, row2d, col2d, ew2d)
    row_flat = row2d.reshape(EP)
    norm_flat = norm2d.reshape(EP)

    x0 = jnp.concatenate([user_emb, item_h], axis=0)
    xs = jnp.concatenate([x0[:, :HD], x0[:, HD:]], axis=0)

    layer_states = []
    for _ in range(NUM_LAYERS):
        xs = _sc_layer(xs, row_flat, col2d, norm_flat)
        layer_states.append(xs)

    x1s, x2s, x3s = layer_states
    final, loss = _tc_final(x0, x1s, x2s, x3s, item_audio_emb, audio_proj_W)
    return (final[:NUM_USERS], final[NUM_USERS:], loss[0, 0])


# speedup vs baseline: 8.7906x; 1.1107x over previous
import functools

import jax
import jax.numpy as jnp
from jax import lax
from jax.experimental import pallas as pl
from jax.experimental.pallas import tpu as pltpu
from jax.experimental.pallas import tpu_sc as plsc

NUM_USERS = 30000
NUM_ITEMS = 20000
N_NODES = 50000
E = 800000
D = 64
HD = 32
NUM_LAYERS = 3

ER = 6656
EP = ER * 128
ROWS_PER_TILE = ER // 16
SUPERS = ROWS_PER_TILE // 32

DEG_ROWS = 512
DEG_TROWS = DEG_ROWS // 16

ITEM_CHUNK = 80
N_ITEM_CHUNKS = NUM_ITEMS // ITEM_CHUNK

_mesh = plsc.VectorSubcoreMesh(core_axis_name="c", subcore_axis_name="s")


def _rsqrt16(x):
    i = plsc.bitcast(x, jnp.int32)
    i = jnp.int32(0x5F3759DF) - (i >> 1)
    y = plsc.bitcast(i, jnp.float32)
    for _ in range(3):
        y = y * (1.5 - 0.5 * x * y * y)
    return jnp.where(x > 0.0, y, 0.0)


def _mlp_body(ef_ref, w1_ref, b1_ref, w2_ref, b2_ref, out_ref):
    ef = ef_ref[...]
    w1 = w1_ref[...]
    h = lax.dot_general(w1, ef, (((1,), (0,)), ((), ())),
                        preferred_element_type=jnp.float32)
    h = jnp.maximum(h + b1_ref[...], 0.0)
    z = lax.dot_general(w2_ref[...], h, (((1,), (0,)), ((), ())),
                        preferred_element_type=jnp.float32)
    out_ref[...] = jax.nn.sigmoid(z + b2_ref[...])


def _edge_mlp(ef_t, w1, b1, w2, b2):
    B = 80000
    grid = E // B
    return pl.pallas_call(
        _mlp_body,
        grid=(grid,),
        in_specs=[
            pl.BlockSpec((5, B), lambda i: (0, i)),
            pl.BlockSpec((32, 5), lambda i: (0, 0)),
            pl.BlockSpec((32, 1), lambda i: (0, 0)),
            pl.BlockSpec((1, 32), lambda i: (0, 0)),
            pl.BlockSpec((1, 1), lambda i: (0, 0)),
        ],
        out_specs=pl.BlockSpec((1, B), lambda i: (0, i)),
        out_shape=jax.ShapeDtypeStruct((1, E), jnp.float32),
    )(ef_t, w1, b1.reshape(32, 1), w2, b2.reshape(1, 1))


def _prep_body(artist_hbm, album_hbm, audio_hbm, aid_hbm, bid_hbm,
               row_hbm, col_hbm, ew_hbm,
               item_hbm, norm_hbm,
               aid_v, bid_v, abuf, bbuf, cbuf,
               deg_tile, idxv, ewv, rowv, nbuf, dbuf, didx,
               deg_sh, dis_sh,
               sem_a, sem_b, sem_c):
    cid = lax.axis_index("c")
    sid = lax.axis_index("s")
    wid = sid * 2 + cid

    def item_chunk(k, _):
        c = wid + 32 * k

        @pl.when(c < N_ITEM_CHUNKS)
        def _():
            base = c * ITEM_CHUNK
            pltpu.sync_copy(aid_hbm.at[pl.ds(base, ITEM_CHUNK)], aid_v)
            pltpu.sync_copy(bid_hbm.at[pl.ds(base, ITEM_CHUNK)], bid_v)
            da = pltpu.async_copy(artist_hbm.at[aid_v], abuf, sem_a)
            db = pltpu.async_copy(album_hbm.at[bid_v], bbuf, sem_b)
            dc = pltpu.async_copy(audio_hbm.at[pl.ds(base, ITEM_CHUNK)], cbuf,
                                  sem_c)
            da.wait()
            db.wait()
            dc.wait()

            def add_row(r, _):
                for kk in range(4):
                    sl = pl.ds(kk * 16, 16)
                    abuf[r, sl] = abuf[r, sl] + bbuf[r, sl] + cbuf[r, sl]
                return ()

            lax.fori_loop(0, ITEM_CHUNK, add_row, (), unroll=4)
            pltpu.sync_copy(abuf, item_hbm.at[pl.ds(base, ITEM_CHUNK)])

        return ()

    lax.fori_loop(0, 8, item_chunk, ())

    def zero_deg(r, _):
        for kk in range(8):
            deg_tile[r, pl.ds(kk * 16, 16)] = jnp.zeros((16,), jnp.float32)
        return ()

    lax.fori_loop(0, DEG_ROWS, zero_deg, (), unroll=2)
    pltpu.sync_copy(deg_tile.at[pl.ds(sid * DEG_TROWS, DEG_TROWS)],
                    deg_sh.at[pl.ds(sid * DEG_TROWS, DEG_TROWS)])

    for j in range(4):
        for kk in range(8):
            didx[j, pl.ds(kk * 16, 16)] = (
                jnp.arange(16, dtype=jnp.int32) + (j * 128 + kk * 16))

    r0_tile = sid * ROWS_PER_TILE

    def deg_super(sb, _):
        r0 = r0_tile + sb * 32
        pltpu.sync_copy(col_hbm.at[pl.ds(r0, 32)], idxv)
        pltpu.sync_copy(ew_hbm.at[pl.ds(r0, 32)], ewv)

        def deg_row(j, _):
            for kk in range(8):
                sl = pl.ds(kk * 16, 16)
                c16 = idxv[j, sl]
                plsc.addupdate_scatter(
                    deg_tile, [c16 >> 7, c16 & 127], ewv[j, sl])
            return ()

        lax.fori_loop(0, 32, deg_row, ())
        return ()

    lax.fori_loop(0, SUPERS, deg_super, ())
    plsc.subcore_barrier()
    for j in range(4):
        pltpu.sync_copy(deg_tile.at[pl.ds(j * 128, 128)],
                        deg_sh.at[didx.at[j]], add=True)
    plsc.subcore_barrier()

    pltpu.sync_copy(deg_sh.at[pl.ds(sid * DEG_TROWS, DEG_TROWS)], dbuf)

    def dis_group(r, _):
        for kk in range(8):
            sl = pl.ds(kk * 16, 16)
            dbuf[r, sl] = _rsqrt16(dbuf[r, sl])
        return ()

    lax.fori_loop(0, DEG_TROWS, dis_group, ())
    pltpu.sync_copy(dbuf, dis_sh.at[pl.ds(sid * DEG_TROWS, DEG_TROWS)])
    plsc.subcore_barrier()

    pltpu.sync_copy(dis_sh, deg_tile)

    def norm_super(sb, _):
        r0 = r0_tile + sb * 32
        pltpu.sync_copy(row_hbm.at[pl.ds(r0, 32)], rowv)
        pltpu.sync_copy(col_hbm.at[pl.ds(r0, 32)], idxv)
        pltpu.sync_copy(ew_hbm.at[pl.ds(r0, 32)], ewv)

        def norm_row(j, _):
            for kk in range(8):
                sl = pl.ds(kk * 16, 16)
                r16 = rowv[j, sl]
                c16 = idxv[j, sl]
                dr = plsc.load_gather(deg_tile, [r16 >> 7, r16 & 127])
                dc = plsc.load_gather(deg_tile, [c16 >> 7, c16 & 127])
                nbuf[j, sl] = dr * ewv[j, sl] * dc
            return ()

        lax.fori_loop(0, 32, norm_row, ())
        pltpu.sync_copy(nbuf, norm_hbm.at[pl.ds(r0, 32)])
        return ()

    lax.fori_loop(0, SUPERS, norm_super, ())


def _sc_prep(artist_emb, album_emb, item_audio_emb, artist_ids, album_ids,
             row2d, col2d, ew2d):
    f32 = jnp.float32
    i32 = jnp.int32
    kern = pl.kernel(
        _prep_body,
        out_type=(
            jax.ShapeDtypeStruct((NUM_ITEMS, D), f32),
            jax.ShapeDtypeStruct((ER, 128), f32),
        ),
        mesh=_mesh,
        compiler_params=pltpu.CompilerParams(needs_layout_passes=False,
                                             use_tc_tiling_on_sc=False),
        scratch_types=[
            pltpu.VMEM((ITEM_CHUNK,), i32),
            pltpu.VMEM((ITEM_CHUNK,), i32),
            pltpu.VMEM((ITEM_CHUNK, D), f32),
            pltpu.VMEM((ITEM_CHUNK, D), f32),
            pltpu.VMEM((ITEM_CHUNK, D), f32),
            pltpu.VMEM((DEG_ROWS, 128), f32),
            pltpu.VMEM((32, 128), i32),
            pltpu.VMEM((32, 128), f32),
            pltpu.VMEM((32, 128), i32),
            pltpu.VMEM((32, 128), f32),
            pltpu.VMEM((DEG_TROWS, 128), f32),
            pltpu.VMEM((4, 128), i32),
            pltpu.VMEM_SHARED((DEG_ROWS, 128), f32),
            pltpu.VMEM_SHARED((DEG_ROWS, 128), f32),
            pltpu.SemaphoreType.DMA,
            pltpu.SemaphoreType.DMA,
            pltpu.SemaphoreType.DMA,
        ],
    )
    return kern(artist_emb, album_emb, item_audio_emb, artist_ids, album_ids,
                row2d, col2d, ew2d)


def _layer_body(xs_hbm, rowf_hbm, col_hbm, normf_hbm, ys_hbm,
                rows, rowi, coli, normv, msg,
                accum, gsem0, gsem1, zsem):
    cid = lax.axis_index("c")
    sid = lax.axis_index("s")
    gsems = (gsem0, gsem1)
    xoff = cid * N_NODES

    def zrow(r, _):
        for kk in range(2):
            msg[r, pl.ds(kk * 16, 16)] = jnp.zeros((16,), jnp.float32)
        return ()
    lax.fori_loop(0, 256, zrow, (), unroll=4)

    zbase = sid * (N_NODES // 16)

    zdescs = []
    for t in range(12):
        zdescs.append(pltpu.async_copy(
            msg, accum.at[pl.ds(zbase + t * 256, 256)], zsem))
    zdescs.append(pltpu.async_copy(
        msg.at[pl.ds(0, 53)], accum.at[pl.ds(zbase + 3072, 53)], zsem))
    for d in zdescs:
        d.wait()
    plsc.subcore_barrier()

    e0_tile = sid * ROWS_PER_TILE * 128

    def _gather(g, b):
        return pltpu.async_copy(
            xs_hbm.at[rowi.at[pl.ds(g * 256, 256)]], rows.at[b], gsems[b])

    mask_hi = jnp.full((16,), jnp.int32(-65536))

    def super_chunk(sb, _):
        e0 = e0_tile + sb * 4096
        r0 = (e0 >> 7)
        pltpu.sync_copy(rowf_hbm.at[pl.ds(e0, 4096)], rowi)
        pltpu.sync_copy(col_hbm.at[pl.ds(r0, 32)], coli)
        pltpu.sync_copy(normf_hbm.at[pl.ds(e0, 4096)], normv)

        def adj_row(j, _):
            rowi[pl.ds(j * 16, 16)] = rowi[pl.ds(j * 16, 16)] + xoff
            return ()

        lax.fori_loop(0, 256, adj_row, (), unroll=8)

        _gather(0, 0)
        _gather(1, 1)
        for g in range(16):
            b = g & 1
            pltpu.make_async_copy(
                xs_hbm.at[rowi.at[pl.ds(g * 256, 256)]], rows.at[b],
                gsems[b]).wait()

            def scale(q, _):
                nv = normv[pl.ds(g * 256 + q * 16, 16)]
                for i in range(16):
                    e = q * 16 + i
                    s = nv[i]
                    iv = rows[b, e, pl.ds(0, 16)]
                    fe = plsc.bitcast(iv << 16, jnp.float32)
                    fo = plsc.bitcast(iv & mask_hi, jnp.float32)
                    msg[e, pl.ds(0, 16)] = fe * s
                    msg[e, pl.ds(16, 16)] = fo * s
                return ()

            lax.fori_loop(0, 16, scale, ())
            for j in range(2):
                pltpu.sync_copy(msg.at[pl.ds(j * 128, 128)],
                                accum.at[coli.at[g * 2 + j]], add=True)
            if g + 2 < 16:
                _gather(g + 2, b)
        return ()

    lax.fori_loop(0, SUPERS, super_chunk, ())
    plsc.subcore_barrier()

    obase = sid * (N_NODES // 16)
    rnd = jnp.full((16,), jnp.int32(0x8000))
    mask_lo = jnp.full((16,), jnp.int32(0xFFFF))

    def out_chunk(t, n):
        pltpu.sync_copy(accum.at[pl.ds(obase + t * 256, n)],
                        msg.at[pl.ds(0, n)])

        def pack_row(r, _):
            a = plsc.bitcast(msg[r, pl.ds(0, 16)], jnp.int32)
            bq = plsc.bitcast(msg[r, pl.ds(16, 16)], jnp.int32)
            lo = ((a + rnd) >> 16) & mask_lo
            hi = (bq + rnd) & mask_hi
            rows[0, r, pl.ds(0, 16)] = lo | hi
            return ()

        lax.fori_loop(0, n, pack_row, (), unroll=2)
        pltpu.sync_copy(rows.at[0, pl.ds(0, n)],
                        ys_hbm.at[pl.ds(xoff + obase + t * 256, n)])
        return ()

    def out_loop(t, _):
        out_chunk(t, 256)
        return ()

    lax.fori_loop(0, 12, out_loop, ())
    out_chunk(12, 53)


def _sc_layer(xs_i, rowf, col2d, normf):
    f32 = jnp.float32
    i32 = jnp.int32
    kern = pl.kernel(
        _layer_body,
        out_type=jax.ShapeDtypeStruct((2 * N_NODES, HD // 2), i32),
        mesh=_mesh,
        compiler_params=pltpu.CompilerParams(needs_layout_passes=False,
                                             use_tc_tiling_on_sc=False),
        scratch_types=[
            pltpu.VMEM((2, 256, HD // 2), i32),
            pltpu.VMEM((4096,), i32),
            pltpu.VMEM((32, 128), i32),
            pltpu.VMEM((4096,), f32),
            pltpu.VMEM((256, HD), f32),
            pltpu.VMEM_SHARED((N_NODES, HD), f32),
            pltpu.SemaphoreType.DMA,
            pltpu.SemaphoreType.DMA,
            pltpu.SemaphoreType.DMA,
        ],
    )
    return kern(xs_i, rowf, col2d, normf)


def _final_body(x0_ref, aL_ref, aR_ref, bL_ref, bR_ref, cL_ref, cR_ref,
                ia_ref, w_ref, out_ref, loss_ref, acc_ref):
    i = pl.program_id(0)
    x0 = x0_ref[...]
    fL = (x0[:, :HD] + aL_ref[...] + bL_ref[...] + cL_ref[...]) * 0.25
    fR = (x0[:, HD:] + aR_ref[...] + bR_ref[...] + cR_ref[...]) * 0.25
    f = jnp.concatenate([fL, fR], axis=1)
    out_ref[...] = f

    @pl.when(i == 0)
    def _():
        acc_ref[0] = 0.0

    @pl.when(i >= 15)
    def _():
        proj = lax.dot_general(ia_ref[...], w_ref[...], (((1,), (1,)), ((), ())),
                               preferred_element_type=jnp.float32)
        d = f - proj
        acc_ref[0] += jnp.sum(d * d)

    @pl.when(i == 24)
    def _():
        loss_ref[...] = jnp.full((1, 1), acc_ref[0] / float(NUM_ITEMS * D),
                                 jnp.float32)


def _tc_final(x0, x1s, x2s, x3s, item_audio_emb, audio_proj_W):
    B = 2000
    nU = NUM_USERS // B
    half = pl.BlockSpec((B, HD), lambda i: (i, 0))

    def rhalf(i):
        return (i + N_NODES // B, 0)

    out, loss = pl.pallas_call(
        _final_body,
        grid=(N_NODES // B,),
        in_specs=[
            pl.BlockSpec((B, D), lambda i: (i, 0)),
            half, pl.BlockSpec((B, HD), rhalf),
            half, pl.BlockSpec((B, HD), rhalf),
            half, pl.BlockSpec((B, HD), rhalf),
            pl.BlockSpec((B, D), lambda i: (jnp.maximum(i - nU, 0), 0)),
            pl.BlockSpec((D, D), lambda i: (0, 0)),
        ],
        out_specs=[
            pl.BlockSpec((B, D), lambda i: (i, 0)),
            pl.BlockSpec((1, 1), lambda i: (0, 0)),
        ],
        out_shape=[
            jax.ShapeDtypeStruct((N_NODES, D), jnp.float32),
            jax.ShapeDtypeStruct((1, 1), jnp.float32),
        ],
        scratch_shapes=[pltpu.SMEM((1,), jnp.float32)],
    )(x0, x1s, x1s, x2s, x2s, x3s, x3s, item_audio_emb, audio_proj_W)
    return out, loss


@jax.jit
def kernel(user_emb, artist_emb, album_emb, item_audio_emb, audio_proj_W,
           mlp_W1, mlp_b1, mlp_W2, mlp_b2, edge_features,
           artist_ids, album_ids, adjusted_edge_index):
    i32 = jnp.int32
    row = adjusted_edge_index[0].astype(i32)
    col = adjusted_edge_index[1].astype(i32)
    aid = artist_ids.astype(i32)
    bid = album_ids.astype(i32)

    ef_t = edge_features.T
    ew = _edge_mlp(ef_t, mlp_W1, mlp_b1, mlp_W2, mlp_b2)

    pad = EP - E
    row2d = jnp.pad(row, (0, pad)).reshape(ER, 128)
    col2d = jnp.pad(col, (0, pad)).reshape(ER, 128)
    ew2d = jnp.pad(ew[0], (0, pad)).reshape(ER, 128)

    item_h, norm2d = _sc_prep(artist_emb, album_emb, item_audio_emb,
                              aid, bid, row2d, col2d, ew2d)
    row_flat = row2d.reshape(EP)
    norm_flat = norm2d.reshape(EP)

    x0 = jnp.concatenate([user_emb, item_h], axis=0)
    xs = jnp.concatenate([x0[:, :HD], x0[:, HD:]], axis=0)
    xs_i = lax.bitcast_convert_type(
        xs.astype(jnp.bfloat16).reshape(2 * N_NODES, HD // 2, 2),
        jnp.int32)

    layer_states = []
    for _ in range(NUM_LAYERS):
        xs_i = _sc_layer(xs_i, row_flat, col2d, norm_flat)
        layer_states.append(
            lax.bitcast_convert_type(xs_i, jnp.bfloat16).reshape(
                2 * N_NODES, HD))

    x1s, x2s, x3s = layer_states
    final, loss = _tc_final(x0, x1s, x2s, x3s, item_audio_emb, audio_proj_W)
    return (final[:NUM_USERS], final[NUM_USERS:], loss[0, 0])
